# Initial kernel scaffold; baseline (speedup 1.0000x reference)
import jax
import jax.numpy as jnp
from jax.experimental import pallas as pl


def kernel(x, table):
    raise NotImplementedError("write your pallas kernel here")


# speedup vs baseline: 1.3853x; 1.3853x over previous
import functools

import jax
import jax.numpy as jnp
from jax import lax
from jax.experimental import pallas as pl
from jax.experimental.pallas import tpu as pltpu
from jax.experimental.pallas import tpu_sc as plsc

N = 100000
D = 128

NC = 2
NS = 16
NW = NC * NS

PER_W = 3120
MAIN = NW * PER_W
TAIL = N - MAIN
CHUNK = 312
NCHUNK = PER_W // CHUNK


def _emb_body(idx_hbm, tbl_hbm, out_hbm,
              i0, i1, r0, r1, ti, tr, g0, g1, o0, o1):
  idx_bufs = (i0, i1)
  row_bufs = (r0, r1)
  gsem = (g0, g1)
  osem = (o0, o1)

  wid = lax.axis_index("s") * NC + lax.axis_index("c")
  base = wid * PER_W

  pltpu.sync_copy(idx_hbm.at[pl.ds(base, CHUNK)], idx_bufs[0])
  gathers = [None, None]
  outs = [None, None]
  gathers[0] = pltpu.async_copy(tbl_hbm.at[idx_bufs[0]], row_bufs[0], gsem[0])

  for g in range(NCHUNK):
    b = g % 2
    nb = (g + 1) % 2
    if g + 1 < NCHUNK:
      if g + 1 >= 2:
        outs[nb].wait()
      pltpu.sync_copy(idx_hbm.at[pl.ds(base + (g + 1) * CHUNK, CHUNK)],
                      idx_bufs[nb])
      gathers[nb] = pltpu.async_copy(tbl_hbm.at[idx_bufs[nb]], row_bufs[nb],
                                     gsem[nb])
    gathers[b].wait()
    outs[b] = pltpu.async_copy(row_bufs[b],
                               out_hbm.at[pl.ds(base + g * CHUNK, CHUNK)],
                               osem[b])
  outs[0].wait()
  outs[1].wait()

  @pl.when(wid == 0)
  def _tail():
    pltpu.sync_copy(idx_hbm.at[pl.ds(MAIN, TAIL)], ti)
    pltpu.async_copy(tbl_hbm.at[ti], tr, g0).wait()
    pltpu.sync_copy(tr, out_hbm.at[pl.ds(MAIN, TAIL)])


@jax.jit
def kernel(x, table):
  idx = jnp.reshape(x, (N,)).astype(jnp.int32)
  tbl = jnp.concatenate([jnp.zeros((1, D), table.dtype), table], axis=0)

  mesh = plsc.VectorSubcoreMesh(core_axis_name="c", subcore_axis_name="s")
  run = pl.kernel(
      _emb_body,
      mesh=mesh,
      out_type=jax.ShapeDtypeStruct((N, D), jnp.float32),
      scratch_types=[
          pltpu.VMEM((CHUNK,), jnp.int32),
          pltpu.VMEM((CHUNK,), jnp.int32),
          pltpu.VMEM((CHUNK, D), jnp.float32),
          pltpu.VMEM((CHUNK, D), jnp.float32),
          pltpu.VMEM((TAIL,), jnp.int32),
          pltpu.VMEM((TAIL, D), jnp.float32),
          pltpu.SemaphoreType.DMA,
          pltpu.SemaphoreType.DMA,
          pltpu.SemaphoreType.DMA,
          pltpu.SemaphoreType.DMA,
      ],
  )
  return run(idx, tbl)


# speedup vs baseline: 4.9095x; 3.5440x over previous
import functools

import jax
import jax.numpy as jnp
from jax import lax
from jax.experimental import pallas as pl
from jax.experimental.pallas import tpu as pltpu
from jax.experimental.pallas import tpu_sc as plsc

N = 100000
D = 128

NC = 2
NS = 16
NW = NC * NS

PER_W = 3120
MAIN = NW * PER_W
TAIL = N - MAIN
CHUNK = 312
NCHUNK = PER_W // CHUNK


def _emb_body(idx_hbm, tbl_hbm, out_hbm,
              tbl_sh, i0, i1, r0, r1, ti, tr, g0, g1, o0, o1):
  idx_bufs = (i0, i1)
  row_bufs = (r0, r1)
  gsem = (g0, g1)
  osem = (o0, o1)

  sid = lax.axis_index("s")
  wid = sid * NC + lax.axis_index("c")
  base = wid * PER_W

  @pl.when(sid == 0)
  def _stage():
    pltpu.sync_copy(tbl_hbm, tbl_sh)

  plsc.subcore_barrier()

  pltpu.sync_copy(idx_hbm.at[pl.ds(base, CHUNK)], idx_bufs[0])
  gathers = [None, None]
  outs = [None, None]
  gathers[0] = pltpu.async_copy(tbl_sh.at[idx_bufs[0]], row_bufs[0], gsem[0])

  for g in range(NCHUNK):
    b = g % 2
    nb = (g + 1) % 2
    if g + 1 < NCHUNK:
      if g + 1 >= 2:
        outs[nb].wait()
      pltpu.sync_copy(idx_hbm.at[pl.ds(base + (g + 1) * CHUNK, CHUNK)],
                      idx_bufs[nb])
      gathers[nb] = pltpu.async_copy(tbl_sh.at[idx_bufs[nb]], row_bufs[nb],
                                     gsem[nb])
    gathers[b].wait()
    outs[b] = pltpu.async_copy(row_bufs[b],
                               out_hbm.at[pl.ds(base + g * CHUNK, CHUNK)],
                               osem[b])
  outs[0].wait()
  outs[1].wait()

  @pl.when(wid == 0)
  def _tail():
    pltpu.sync_copy(idx_hbm.at[pl.ds(MAIN, TAIL)], ti)
    pltpu.async_copy(tbl_sh.at[ti], tr, g0).wait()
    pltpu.sync_copy(tr, out_hbm.at[pl.ds(MAIN, TAIL)])


@jax.jit
def kernel(x, table):
  idx = jnp.reshape(x, (N,)).astype(jnp.int32)
  tbl = jnp.concatenate([jnp.zeros((1, D), table.dtype), table], axis=0)

  mesh = plsc.VectorSubcoreMesh(core_axis_name="c", subcore_axis_name="s")
  run = pl.kernel(
      _emb_body,
      mesh=mesh,
      out_type=jax.ShapeDtypeStruct((N, D), jnp.float32),
      scratch_types=[
          pltpu.VMEM_SHARED((94, D), jnp.float32),
          pltpu.VMEM((CHUNK,), jnp.int32),
          pltpu.VMEM((CHUNK,), jnp.int32),
          pltpu.VMEM((CHUNK, D), jnp.float32),
          pltpu.VMEM((CHUNK, D), jnp.float32),
          pltpu.VMEM((TAIL,), jnp.int32),
          pltpu.VMEM((TAIL, D), jnp.float32),
          pltpu.SemaphoreType.DMA,
          pltpu.SemaphoreType.DMA,
          pltpu.SemaphoreType.DMA,
          pltpu.SemaphoreType.DMA,
      ],
  )
  return run(idx, tbl)


# speedup vs baseline: 5.0392x; 1.0264x over previous
import functools

import jax
import jax.numpy as jnp
from jax import lax
from jax.experimental import pallas as pl
from jax.experimental.pallas import tpu as pltpu
from jax.experimental.pallas import tpu_sc as plsc

N = 100000
D = 128

NC = 2
NS = 16
NW = NC * NS

PER_W = 3120
MAIN = NW * PER_W
TAIL = N - MAIN
CHUNK = 312
NCHUNK = PER_W // CHUNK


def _emb_body(idx_hbm, tbl_hbm, out_hbm,
              tbl_sh, i0, i1, r0, r1, ti, tr, g0, g1, o0, o1):
  idx_bufs = (i0, i1)
  row_bufs = (r0, r1)
  gsem = (g0, g1)
  osem = (o0, o1)

  sid = lax.axis_index("s")
  wid = sid * NC + lax.axis_index("c")
  base = wid * PER_W

  @pl.when(sid == 0)
  def _stage():
    pltpu.sync_copy(tbl_hbm, tbl_sh.at[pl.ds(1, 93)])

  plsc.subcore_barrier()

  pltpu.sync_copy(idx_hbm.at[pl.ds(base, CHUNK)], idx_bufs[0])
  gathers = [None, None]
  outs = [None, None]
  gathers[0] = pltpu.async_copy(tbl_sh.at[idx_bufs[0]], row_bufs[0], gsem[0])

  for g in range(NCHUNK):
    b = g % 2
    nb = (g + 1) % 2
    if g + 1 < NCHUNK:
      if g + 1 >= 2:
        outs[nb].wait()
      pltpu.sync_copy(idx_hbm.at[pl.ds(base + (g + 1) * CHUNK, CHUNK)],
                      idx_bufs[nb])
      gathers[nb] = pltpu.async_copy(tbl_sh.at[idx_bufs[nb]], row_bufs[nb],
                                     gsem[nb])
    gathers[b].wait()
    outs[b] = pltpu.async_copy(row_bufs[b],
                               out_hbm.at[pl.ds(base + g * CHUNK, CHUNK)],
                               osem[b])
  outs[0].wait()
  outs[1].wait()

  @pl.when(wid == 0)
  def _tail():
    pltpu.sync_copy(idx_hbm.at[pl.ds(MAIN, TAIL)], ti)
    pltpu.async_copy(tbl_sh.at[ti], tr, g0).wait()
    pltpu.sync_copy(tr, out_hbm.at[pl.ds(MAIN, TAIL)])


@jax.jit
def kernel(x, table):
  idx = jnp.reshape(x, (N,)).astype(jnp.int32)

  mesh = plsc.VectorSubcoreMesh(core_axis_name="c", subcore_axis_name="s")
  run = pl.kernel(
      _emb_body,
      mesh=mesh,
      out_type=jax.ShapeDtypeStruct((N, D), jnp.float32),
      scratch_types=[
          pltpu.VMEM_SHARED((94, D), jnp.float32),
          pltpu.VMEM((CHUNK,), jnp.int32),
          pltpu.VMEM((CHUNK,), jnp.int32),
          pltpu.VMEM((CHUNK, D), jnp.float32),
          pltpu.VMEM((CHUNK, D), jnp.float32),
          pltpu.VMEM((TAIL,), jnp.int32),
          pltpu.VMEM((TAIL, D), jnp.float32),
          pltpu.SemaphoreType.DMA,
          pltpu.SemaphoreType.DMA,
          pltpu.SemaphoreType.DMA,
          pltpu.SemaphoreType.DMA,
      ],
  )
  return run(idx, table)


# speedup vs baseline: 5.1823x; 1.0284x over previous
import functools

import jax
import jax.numpy as jnp
from jax import lax
from jax.experimental import pallas as pl
from jax.experimental.pallas import tpu as pltpu
from jax.experimental.pallas import tpu_sc as plsc

N = 100000
D = 128

NC = 2
NS = 16
NW = NC * NS

PER_W = 3120
EXTRA = 16
NEXTRA = 10
CHUNK = 312
NCHUNK = PER_W // CHUNK


def _emb_body(idx_hbm, tbl_hbm, out_hbm,
              tbl_sh, i0, i1, r0, r1, ti, tr, g0, g1, o0, o1):
  idx_bufs = (i0, i1)
  row_bufs = (r0, r1)
  gsem = (g0, g1)
  osem = (o0, o1)

  sid = lax.axis_index("s")
  wid = sid * NC + lax.axis_index("c")
  base = wid * PER_W + jnp.minimum(wid, NEXTRA) * EXTRA

  @pl.when(sid == 0)
  def _stage():
    pltpu.sync_copy(tbl_hbm, tbl_sh.at[pl.ds(1, 93)])

  plsc.subcore_barrier()

  pltpu.sync_copy(idx_hbm.at[pl.ds(base, CHUNK)], idx_bufs[0])
  gathers = [None, None]
  outs = [None, None]
  gathers[0] = pltpu.async_copy(tbl_sh.at[idx_bufs[0]], row_bufs[0], gsem[0])

  for g in range(NCHUNK):
    b = g % 2
    nb = (g + 1) % 2
    if g + 1 < NCHUNK:
      if g + 1 >= 2:
        outs[nb].wait()
      pltpu.sync_copy(idx_hbm.at[pl.ds(base + (g + 1) * CHUNK, CHUNK)],
                      idx_bufs[nb])
      gathers[nb] = pltpu.async_copy(tbl_sh.at[idx_bufs[nb]], row_bufs[nb],
                                     gsem[nb])
    gathers[b].wait()
    outs[b] = pltpu.async_copy(row_bufs[b],
                               out_hbm.at[pl.ds(base + g * CHUNK, CHUNK)],
                               osem[b])
  outs[0].wait()
  outs[1].wait()

  @pl.when(wid < NEXTRA)
  def _tail():
    tbase = base + PER_W
    pltpu.sync_copy(idx_hbm.at[pl.ds(tbase, EXTRA)], ti)
    pltpu.async_copy(tbl_sh.at[ti], tr, g0).wait()
    pltpu.sync_copy(tr, out_hbm.at[pl.ds(tbase, EXTRA)])


@jax.jit
def kernel(x, table):
  idx = jnp.reshape(x, (N,)).astype(jnp.int32)

  mesh = plsc.VectorSubcoreMesh(core_axis_name="c", subcore_axis_name="s")
  run = pl.kernel(
      _emb_body,
      mesh=mesh,
      out_type=jax.ShapeDtypeStruct((N, D), jnp.float32),
      scratch_types=[
          pltpu.VMEM_SHARED((94, D), jnp.float32),
          pltpu.VMEM((CHUNK,), jnp.int32),
          pltpu.VMEM((CHUNK,), jnp.int32),
          pltpu.VMEM((CHUNK, D), jnp.float32),
          pltpu.VMEM((CHUNK, D), jnp.float32),
          pltpu.VMEM((EXTRA,), jnp.int32),
          pltpu.VMEM((EXTRA, D), jnp.float32),
          pltpu.SemaphoreType.DMA,
          pltpu.SemaphoreType.DMA,
          pltpu.SemaphoreType.DMA,
          pltpu.SemaphoreType.DMA,
      ],
  )
  return run(idx, table)
